# baseline (speedup 1.0000x reference)
import functools

import jax
import jax.numpy as jnp
from jax import lax
from jax.experimental import pallas as pl
from jax.experimental.pallas import tpu as pltpu
from jax.experimental.pallas import tpu_sc as plsc

H = 64
BM = 512
NC, NS = 2, 16
NW = NC * NS


def _pick_ch_epad(e):
    ch = 128
    epad = -(-e // (64 * ch)) * (64 * ch)
    return ch, epad


def _pad_rows(x, mult):
    n = x.shape[0]
    pad = (-n) % mult
    if pad == 0:
        return x
    return jnp.pad(x, ((0, pad),) + ((0, 0),) * (x.ndim - 1))


def _silu(x):
    return x * jax.nn.sigmoid(x)


def _layer_norm(z, g, b):
    mu = jnp.mean(z, axis=-1, keepdims=True)
    var = jnp.mean((z - mu) * (z - mu), axis=-1, keepdims=True)
    return (z - mu) * jax.lax.rsqrt(var + 1e-5) * g + b


def _proj_pair_body(x_ref, wa_ref, wb_ref, o_ref):
    x = x_ref[...]
    o_ref[:, :H] = jnp.dot(x, wa_ref[...], preferred_element_type=jnp.float32)
    o_ref[:, H:] = jnp.dot(x, wb_ref[...], preferred_element_type=jnp.float32)


def _proj_pair(x, wa, wb):
    xp = _pad_rows(x, BM)
    m = xp.shape[0]
    grid = (m // BM,)
    blk = pl.BlockSpec((BM, H), lambda i: (i, 0))
    full = pl.BlockSpec((H, H), lambda i: (0, 0))
    return pl.pallas_call(
        _proj_pair_body,
        grid=grid,
        in_specs=[blk, full, full],
        out_specs=pl.BlockSpec((BM, 2 * H), lambda i: (i, 0)),
        out_shape=jax.ShapeDtypeStruct((m, 2 * H), jnp.float32),
    )(xp, wa, wb)


def _edge_mlp_body(ea_ref, g_ref, w1_ref, b1_ref, w2_ref, b2_ref,
                   w3_ref, b3_ref, gam_ref, bet_ref, o_ref):
    x = jnp.dot(ea_ref[...], w1_ref[...], preferred_element_type=jnp.float32)
    x = _silu(x + b1_ref[...] + g_ref[...])
    x = _silu(jnp.dot(x, w2_ref[...], preferred_element_type=jnp.float32) + b2_ref[...])
    z = jnp.dot(x, w3_ref[...], preferred_element_type=jnp.float32) + b3_ref[...]
    o_ref[...] = _layer_norm(z, gam_ref[...], bet_ref[...])


def _edge_mlp(ea_p, g_p, w1e, b1, w2, b2, w3, b3, gam, bet):
    m = ea_p.shape[0]
    grid = (m // BM,)
    blk = pl.BlockSpec((BM, H), lambda i: (i, 0))
    wfull = pl.BlockSpec((H, H), lambda i: (0, 0))
    vfull = pl.BlockSpec((1, H), lambda i: (0, 0))
    return pl.pallas_call(
        _edge_mlp_body,
        grid=grid,
        in_specs=[blk, blk, wfull, vfull, wfull, vfull, wfull, vfull,
                  vfull, vfull],
        out_specs=blk,
        out_shape=jax.ShapeDtypeStruct((m, H), jnp.float32),
    )(ea_p, g_p, w1e, b1.reshape(1, H), w2, b2.reshape(1, H),
      w3, b3.reshape(1, H), gam.reshape(1, H), bet.reshape(1, H))


def _node_mlp_body(*refs, alpha, final_ln):
    (rec_ref, ag_ref, wr_ref, wa_ref, b1_ref, w2_ref, b2_ref,
     w3_ref, b3_ref, gam_ref, bet_ref) = refs[:11]
    o_ref = refs[-1]
    rec = rec_ref[...]
    x = (jnp.dot(rec, wr_ref[...], preferred_element_type=jnp.float32)
         + jnp.dot(ag_ref[...], wa_ref[...], preferred_element_type=jnp.float32)
         + b1_ref[...])
    x = _silu(x)
    x = _silu(jnp.dot(x, w2_ref[...], preferred_element_type=jnp.float32) + b2_ref[...])
    z = jnp.dot(x, w3_ref[...], preferred_element_type=jnp.float32) + b3_ref[...]
    upd = _layer_norm(z, gam_ref[...], bet_ref[...])
    out = alpha * rec + upd
    if final_ln:
        resid_ref, g2_ref, b2ln_ref = refs[11:14]
        out = _layer_norm(out + resid_ref[...], g2_ref[...], b2ln_ref[...])
    o_ref[...] = out


def _node_mlp(rec_p, aggr_p, wr, wa, b1, w2, b2, w3, b3, gam, bet,
              alpha, resid_p=None, norm=None):
    m = rec_p.shape[0]
    grid = (m // BM,)
    blk = pl.BlockSpec((BM, H), lambda i: (i, 0))
    wfull = pl.BlockSpec((H, H), lambda i: (0, 0))
    vfull = pl.BlockSpec((1, H), lambda i: (0, 0))
    final_ln = resid_p is not None
    in_specs = [blk, blk, wfull, wfull, vfull, wfull, vfull, wfull, vfull,
                vfull, vfull]
    args = [rec_p, aggr_p, wr, wa, b1.reshape(1, H), w2, b2.reshape(1, H),
            w3, b3.reshape(1, H), gam.reshape(1, H), bet.reshape(1, H)]
    if final_ln:
        in_specs += [blk, vfull, vfull]
        args += [resid_p, norm[0].reshape(1, H), norm[1].reshape(1, H)]
    body = functools.partial(_node_mlp_body, alpha=alpha, final_ln=final_ln)

    return pl.pallas_call(
        body,
        grid=grid,
        in_specs=in_specs,
        out_specs=blk,
        out_shape=jax.ShapeDtypeStruct((m, H), jnp.float32),
    )(*args)


def _sc_gather_sum(s_tab, r_tab, src2d, dst2d, ch):
    epad = src2d.shape[0] * 128
    assert ch == 128
    chunk128 = epad // NW // 128
    n_pairs = chunk128 // 2
    mesh = plsc.VectorSubcoreMesh(core_axis_name="c", subcore_axis_name="s")

    @functools.partial(
        pl.kernel, mesh=mesh,
        out_type=jax.ShapeDtypeStruct((epad, H), jnp.float32),
        scratch_types=[
            *([pltpu.VMEM((1, 128), jnp.int32)] * 2),
            *([pltpu.VMEM((1, 128), jnp.int32)] * 2),
            *([pltpu.VMEM((ch, 2 * H), jnp.float32)] * 2),
            *([pltpu.VMEM((ch, 2 * H), jnp.float32)] * 2),
            *([pltpu.VMEM((ch, H), jnp.float32)] * 2),
            pltpu.SemaphoreType.DMA,
            pltpu.SemaphoreType.DMA,
        ],
    )
    def kern(s_hbm, r_hbm, src_hbm, dst_hbm, g_hbm, *scr):
        si_v, di_v = scr[0:2], scr[2:4]
        s_v, r_v = scr[4:6], scr[6:8]
        g_v = scr[8:10]
        sems = scr[10:12]
        wid = lax.axis_index("s") * NC + lax.axis_index("c")
        row0 = wid * chunk128

        def start(r128, b):
            pltpu.sync_copy(src_hbm.at[pl.ds(r128, 1)], si_v[b])
            pltpu.sync_copy(dst_hbm.at[pl.ds(r128, 1)], di_v[b])
            return (
                pltpu.async_copy(s_hbm.at[si_v[b].at[0]], s_v[b], sems[b]),
                pltpu.async_copy(r_hbm.at[di_v[b].at[0]], r_v[b], sems[b]),
            )

        def finish(cps, r128, b):
            for cp in cps:
                cp.wait()

            def row_body(r, c2):
                for cc in range(4):
                    g_v[b][r, pl.ds(cc * 16, 16)] = (
                        s_v[b][r, pl.ds(cc * 16, 16)]
                        + r_v[b][r, pl.ds(H + cc * 16, 16)])
                return c2

            lax.fori_loop(0, ch, row_body, 0)
            pltpu.sync_copy(g_v[b], g_hbm.at[pl.ds(r128 * 128, ch)])

        def body(i2, carry):
            r128 = row0 + i2 * 2
            cps0 = start(r128, 0)
            cps1 = start(r128 + 1, 1)
            finish(cps0, r128, 0)
            finish(cps1, r128 + 1, 1)
            return carry

        lax.fori_loop(0, n_pairs, body, 0)

    return kern(s_tab, r_tab, src2d, dst2d)


def _sc_segsum(m_p, dst2d, np_rows, ch):
    epad = dst2d.shape[0] * 128
    k = ch // 128
    chunk128 = epad // NW // 128
    n_ch = chunk128 // k
    pc = 1
    while (np_rows // (NC * pc)) > 24000 or np_rows % (NC * pc * 16) != 0:
        pc += 1
    ps = np_rows // (NC * pc)
    acc_rows = ps + 16
    zrows = acc_rows // NS
    orows = ps // NS
    mesh = plsc.VectorSubcoreMesh(core_axis_name="c", subcore_axis_name="s")

    @functools.partial(
        pl.kernel, mesh=mesh,
        out_type=jax.ShapeDtypeStruct((np_rows, H), jnp.float32),
        scratch_types=[
            pltpu.VMEM((k, 128), jnp.int32),
            *([pltpu.VMEM((128,), jnp.int32)] * k),
            pltpu.VMEM((ch, H), jnp.float32),
            pltpu.VMEM_SHARED((acc_rows, H), jnp.float32),
            pltpu.SemaphoreType.DMA,
        ],
    )
    def kern(m_hbm, dst_hbm, out_hbm, di_v, *rest):
        ai_refs = rest[:k]
        v_v, acc_sh, sem = rest[k:]
        cid = lax.axis_index("c")
        sid = lax.axis_index("s")
        wid = sid * NC + cid
        row0 = wid * chunk128

        for p in range(pc):
            base = (cid * pc + p) * ps

            def zbody(r, c):
                for cc in range(4):
                    v_v[r, pl.ds(cc * 16, 16)] = jnp.zeros((16,), jnp.float32)
                return c
            lax.fori_loop(0, ch, zbody, 0)
            off = 0
            while off < zrows:
                sz = min(ch, zrows - off)
                pltpu.sync_copy(v_v.at[pl.ds(0, sz)],
                                acc_sh.at[pl.ds(sid * zrows + off, sz)])
                off += sz
            plsc.subcore_barrier()

            def body(t, carry):
                r128 = row0 + t * k
                pltpu.sync_copy(dst_hbm.at[pl.ds(r128, k)], di_v)
                pltpu.sync_copy(m_hbm.at[pl.ds(r128 * 128, ch)], v_v)
                for j in range(k):
                    for cc in range(8):
                        v = di_v[j, pl.ds(cc * 16, 16)]
                        loc = v - jnp.full((16,), base, jnp.int32)
                        ok = (loc >= 0) & (loc < ps)
                        ai_refs[j][pl.ds(cc * 16, 16)] = jnp.where(
                            ok, loc, jnp.full((16,), ps, jnp.int32))
                for j in range(k):
                    pltpu.sync_copy(v_v.at[pl.ds(j * 128, 128)],
                                    acc_sh.at[ai_refs[j]], add=True)
                return carry

            lax.fori_loop(0, n_ch, body, 0)
            plsc.subcore_barrier()

            off = 0
            while off < orows:
                sz = min(ch, orows - off)
                pltpu.sync_copy(acc_sh.at[pl.ds(sid * orows + off, sz)],
                                out_hbm.at[pl.ds(base + sid * orows + off, sz)])
                off += sz
            if p + 1 < pc:
                plsc.subcore_barrier()

    return kern(m_p, dst2d)


def _prep_edges(ei, ea):
    e = ea.shape[0]
    ch, epad = _pick_ch_epad(e)
    src = ei[0].astype(jnp.int32)
    dst = ei[1].astype(jnp.int32)
    src2d = jnp.pad(src, (0, epad - e)).reshape(epad // 128, 128)
    dst2d = jnp.pad(dst, (0, epad - e)).reshape(epad // 128, 128)
    dst2d_s = jnp.pad(dst, (0, epad - e),
                      constant_values=1 << 29).reshape(epad // 128, 128)
    ea_p = _pad_rows(ea, epad)
    return dict(e=e, ch=ch, epad=epad, src2d=src2d, dst2d=dst2d,
                dst2d_s=dst2d_s, ea_p=ea_p, dst=dst)


def _inet(p, send_p, rec_p, ep, alpha, resid_p=None, norm=None):
    w1, b1 = p['edge_mlp']['linears'][0]
    w2, b2 = p['edge_mlp']['linears'][1]
    w3, b3 = p['edge_mlp']['linears'][2]
    gam, bet = p['edge_mlp']['ln']
    if send_p is rec_p:
        s_tab = r_tab = _proj_pair(send_p, w1[H:2 * H], w1[2 * H:])
    else:
        s_tab = _proj_pair(send_p, w1[H:2 * H], w1[H:2 * H])
        r_tab = _proj_pair(rec_p, w1[2 * H:], w1[2 * H:])
    g = _sc_gather_sum(s_tab, r_tab, ep['src2d'], ep['dst2d'], ep['ch'])
    m_edges = _edge_mlp(ep['ea_p'], g, w1[:H], b1, w2, b2, w3, b3,
                        gam, bet)
    aggr_p = jax.ops.segment_sum(m_edges[:ep['e']], ep['dst'],
                                 num_segments=rec_p.shape[0])
    wn1, bn1 = p['node_mlp']['linears'][0]
    wn2, bn2 = p['node_mlp']['linears'][1]
    wn3, bn3 = p['node_mlp']['linears'][2]
    ngam, nbet = p['node_mlp']['ln']
    return _node_mlp(rec_p, aggr_p, wn1[:H], wn1[H:], bn1, wn2, bn2,
                     wn3, bn3, ngam, nbet, alpha, resid_p, norm)


NUM_LEVELS_K = 3
NUM_STEPS_K = 2


def kernel(mesh_features_list, mesh_edge_index_list, mesh_edge_attr_list,
           up_edge_index_list, up_edge_attr_list, down_edge_index_list,
           down_edge_attr_list, params):
    n_sizes = [f.shape[0] for f in mesh_features_list]
    feats = [_pad_rows(f, BM) for f in mesh_features_list]
    for level in range(NUM_LEVELS_K):
        residual = feats[level]
        ep = _prep_edges(mesh_edge_index_list[level],
                         mesh_edge_attr_list[level])
        for step in range(NUM_STEPS_K):
            last = step == NUM_STEPS_K - 1
            feats[level] = _inet(
                params['intra'][level][step], feats[level], feats[level],
                ep, alpha=1.0,
                resid_p=residual if last else None,
                norm=params['norms'][level] if last else None)
    for level in range(NUM_LEVELS_K - 1):
        ep = _prep_edges(up_edge_index_list[level], up_edge_attr_list[level])
        feats[level + 1] = _inet(
            params['up'][level], feats[level], feats[level + 1], ep,
            alpha=2.0)
    for level in reversed(range(NUM_LEVELS_K - 1)):
        ep = _prep_edges(down_edge_index_list[level],
                         down_edge_attr_list[level])
        feats[level] = _inet(
            params['down'][level], feats[level + 1], feats[level], ep,
            alpha=2.0)
    return tuple(feats[l][:n_sizes[l]] for l in range(NUM_LEVELS_K))


# speedup vs baseline: 1.0885x; 1.0885x over previous
import functools

import jax
import jax.numpy as jnp
from jax import lax
from jax.experimental import pallas as pl
from jax.experimental.pallas import tpu as pltpu
from jax.experimental.pallas import tpu_sc as plsc

H = 64
BM = 512
NC, NS = 2, 16
NW = NC * NS


def _pick_ch_epad(e):
    ch = 128
    epad = -(-e // 4096) * 4096
    return ch, epad


def _pad_rows(x, mult):
    n = x.shape[0]
    pad = (-n) % mult
    if pad == 0:
        return x
    return jnp.pad(x, ((0, pad),) + ((0, 0),) * (x.ndim - 1))


def _silu(x):
    return x * jax.nn.sigmoid(x)


def _layer_norm(z, g, b):
    mu = jnp.mean(z, axis=-1, keepdims=True)
    var = jnp.mean((z - mu) * (z - mu), axis=-1, keepdims=True)
    return (z - mu) * jax.lax.rsqrt(var + 1e-5) * g + b


def _proj_pair_body(x_ref, wa_ref, wb_ref, o_ref):
    x = x_ref[...]
    o_ref[:, :H] = jnp.dot(x, wa_ref[...], preferred_element_type=jnp.float32)
    o_ref[:, H:] = jnp.dot(x, wb_ref[...], preferred_element_type=jnp.float32)


def _proj_pair(x, wa, wb):
    xp = _pad_rows(x, BM)
    m = xp.shape[0]
    grid = (m // BM,)
    blk = pl.BlockSpec((BM, H), lambda i: (i, 0))
    full = pl.BlockSpec((H, H), lambda i: (0, 0))
    return pl.pallas_call(
        _proj_pair_body,
        grid=grid,
        in_specs=[blk, full, full],
        out_specs=pl.BlockSpec((BM, 2 * H), lambda i: (i, 0)),
        out_shape=jax.ShapeDtypeStruct((m, 2 * H), jnp.float32),
    )(xp, wa, wb)


def _edge_mlp_body(ea_ref, g_ref, w1_ref, b1_ref, w2_ref, b2_ref,
                   w3_ref, b3_ref, gam_ref, bet_ref, o_ref):
    x = jnp.dot(ea_ref[...], w1_ref[...], preferred_element_type=jnp.float32)
    x = _silu(x + b1_ref[...] + g_ref[...])
    x = _silu(jnp.dot(x, w2_ref[...], preferred_element_type=jnp.float32) + b2_ref[...])
    z = jnp.dot(x, w3_ref[...], preferred_element_type=jnp.float32) + b3_ref[...]
    o_ref[...] = _layer_norm(z, gam_ref[...], bet_ref[...])


def _edge_mlp(ea_p, g_p, w1e, b1, w2, b2, w3, b3, gam, bet):
    m = ea_p.shape[0]
    grid = (m // BM,)
    blk = pl.BlockSpec((BM, H), lambda i: (i, 0))
    wfull = pl.BlockSpec((H, H), lambda i: (0, 0))
    vfull = pl.BlockSpec((1, H), lambda i: (0, 0))
    return pl.pallas_call(
        _edge_mlp_body,
        grid=grid,
        in_specs=[blk, blk, wfull, vfull, wfull, vfull, wfull, vfull,
                  vfull, vfull],
        out_specs=blk,
        out_shape=jax.ShapeDtypeStruct((m, H), jnp.float32),
    )(ea_p, g_p, w1e, b1.reshape(1, H), w2, b2.reshape(1, H),
      w3, b3.reshape(1, H), gam.reshape(1, H), bet.reshape(1, H))


def _node_mlp_body(*refs, alpha, final_ln):
    (rec_ref, ag_ref, wr_ref, wa_ref, b1_ref, w2_ref, b2_ref,
     w3_ref, b3_ref, gam_ref, bet_ref) = refs[:11]
    o_ref = refs[-1]
    rec = rec_ref[...]
    x = (jnp.dot(rec, wr_ref[...], preferred_element_type=jnp.float32)
         + jnp.dot(ag_ref[...], wa_ref[...], preferred_element_type=jnp.float32)
         + b1_ref[...])
    x = _silu(x)
    x = _silu(jnp.dot(x, w2_ref[...], preferred_element_type=jnp.float32) + b2_ref[...])
    z = jnp.dot(x, w3_ref[...], preferred_element_type=jnp.float32) + b3_ref[...]
    upd = _layer_norm(z, gam_ref[...], bet_ref[...])
    out = alpha * rec + upd
    if final_ln:
        resid_ref, g2_ref, b2ln_ref = refs[11:14]
        out = _layer_norm(out + resid_ref[...], g2_ref[...], b2ln_ref[...])
    o_ref[...] = out


def _node_mlp(rec_p, aggr_p, wr, wa, b1, w2, b2, w3, b3, gam, bet,
              alpha, resid_p=None, norm=None):
    m = rec_p.shape[0]
    grid = (m // BM,)
    blk = pl.BlockSpec((BM, H), lambda i: (i, 0))
    wfull = pl.BlockSpec((H, H), lambda i: (0, 0))
    vfull = pl.BlockSpec((1, H), lambda i: (0, 0))
    final_ln = resid_p is not None
    in_specs = [blk, blk, wfull, wfull, vfull, wfull, vfull, wfull, vfull,
                vfull, vfull]
    args = [rec_p, aggr_p, wr, wa, b1.reshape(1, H), w2, b2.reshape(1, H),
            w3, b3.reshape(1, H), gam.reshape(1, H), bet.reshape(1, H)]
    if final_ln:
        in_specs += [blk, vfull, vfull]
        args += [resid_p, norm[0].reshape(1, H), norm[1].reshape(1, H)]
    body = functools.partial(_node_mlp_body, alpha=alpha, final_ln=final_ln)

    return pl.pallas_call(
        body,
        grid=grid,
        in_specs=in_specs,
        out_specs=blk,
        out_shape=jax.ShapeDtypeStruct((m, H), jnp.float32),
    )(*args)


def _sc_gather_sum(s_tab, r_tab, src2d, dst2d, ch):
    epad = src2d.shape[0] * 128
    assert ch == 128
    ib = 16
    n_blocks = epad // 4096
    base_n, extra = divmod(n_blocks, NW)
    mesh = plsc.VectorSubcoreMesh(core_axis_name="c", subcore_axis_name="s")

    @functools.partial(
        pl.kernel, mesh=mesh,
        out_type=jax.ShapeDtypeStruct((epad, H), jnp.float32),
        scratch_types=[
            pltpu.VMEM((2 * ib, 128), jnp.int32),
            pltpu.VMEM((2 * ib, 128), jnp.int32),
            *([pltpu.VMEM((ch, 2 * H), jnp.float32)] * 2),
            *([pltpu.VMEM((ch, 2 * H), jnp.float32)] * 2),
            *([pltpu.VMEM((ch, H), jnp.float32)] * 2),
            pltpu.SemaphoreType.DMA,
            pltpu.SemaphoreType.DMA,
            pltpu.SemaphoreType.DMA,
        ],
    )
    def kern(s_hbm, r_hbm, src_hbm, dst_hbm, g_hbm, *scr):
        si_blk, di_blk = scr[0], scr[1]
        s_v, r_v = scr[2:4], scr[4:6]
        g_v = scr[6:8]
        sem0, sem1, osem = scr[8:11]
        sems = (sem0, sem1)
        wid = lax.axis_index("s") * NC + lax.axis_index("c")
        n_w = base_n + jnp.where(wid < extra, 1, 0)

        def compute(b):
            def row_body(r, c2):
                for cc in range(4):
                    g_v[b][r, pl.ds(cc * 16, 16)] = (
                        s_v[b][r, pl.ds(cc * 16, 16)]
                        + r_v[b][r, pl.ds(H + cc * 16, 16)])
                return c2
            lax.fori_loop(0, ch, row_body, 0)

        def blk_body(blk128, npair):
            cpi = (pltpu.async_copy(src_hbm.at[pl.ds(blk128, 2 * npair)],
                                    si_blk, sem0),
                   pltpu.async_copy(dst_hbm.at[pl.ds(blk128, 2 * npair)],
                                    di_blk, sem0))
            for cp in cpi:
                cp.wait()

            def pair(p, carry):
                cps = []
                for b in range(2):
                    cps.append((pltpu.async_copy(
                        s_hbm.at[si_blk.at[2 * p + b]], s_v[b], sems[b]),
                        pltpu.async_copy(
                        r_hbm.at[di_blk.at[2 * p + b]], r_v[b], sems[b])))
                outs = []
                for b in range(2):
                    for cp in cps[b]:
                        cp.wait()
                    compute(b)
                    outs.append(pltpu.async_copy(
                        g_v[b],
                        g_hbm.at[pl.ds((blk128 + 2 * p + b) * 128, ch)],
                        osem))
                for cp in outs:
                    cp.wait()
                return carry

            lax.fori_loop(0, npair, pair, 0)

        def blocks(i, carry):
            blk_body((i * NW + wid) * 2 * ib, ib)
            return carry

        lax.fori_loop(0, n_w, blocks, 0)

    return kern(s_tab, r_tab, src2d, dst2d)


def _sc_segsum(m_p, dst2d, np_rows, ch):
    epad = dst2d.shape[0] * 128
    k = ch // 128
    chunk128 = epad // NW // 128
    n_ch = chunk128 // k
    pc = 1
    while (np_rows // (NC * pc)) > 24000 or np_rows % (NC * pc * 16) != 0:
        pc += 1
    ps = np_rows // (NC * pc)
    acc_rows = ps + 16
    zrows = acc_rows // NS
    orows = ps // NS
    mesh = plsc.VectorSubcoreMesh(core_axis_name="c", subcore_axis_name="s")

    @functools.partial(
        pl.kernel, mesh=mesh,
        out_type=jax.ShapeDtypeStruct((np_rows, H), jnp.float32),
        scratch_types=[
            pltpu.VMEM((k, 128), jnp.int32),
            *([pltpu.VMEM((128,), jnp.int32)] * k),
            pltpu.VMEM((ch, H), jnp.float32),
            pltpu.VMEM_SHARED((acc_rows, H), jnp.float32),
            pltpu.SemaphoreType.DMA,
        ],
    )
    def kern(m_hbm, dst_hbm, out_hbm, di_v, *rest):
        ai_refs = rest[:k]
        v_v, acc_sh, sem = rest[k:]
        cid = lax.axis_index("c")
        sid = lax.axis_index("s")
        wid = sid * NC + cid
        row0 = wid * chunk128

        for p in range(pc):
            base = (cid * pc + p) * ps

            def zbody(r, c):
                for cc in range(4):
                    v_v[r, pl.ds(cc * 16, 16)] = jnp.zeros((16,), jnp.float32)
                return c
            lax.fori_loop(0, ch, zbody, 0)
            off = 0
            while off < zrows:
                sz = min(ch, zrows - off)
                pltpu.sync_copy(v_v.at[pl.ds(0, sz)],
                                acc_sh.at[pl.ds(sid * zrows + off, sz)])
                off += sz
            plsc.subcore_barrier()

            def body(t, carry):
                r128 = row0 + t * k
                pltpu.sync_copy(dst_hbm.at[pl.ds(r128, k)], di_v)
                pltpu.sync_copy(m_hbm.at[pl.ds(r128 * 128, ch)], v_v)
                for j in range(k):
                    for cc in range(8):
                        v = di_v[j, pl.ds(cc * 16, 16)]
                        loc = v - jnp.full((16,), base, jnp.int32)
                        ok = (loc >= 0) & (loc < ps)
                        ai_refs[j][pl.ds(cc * 16, 16)] = jnp.where(
                            ok, loc, jnp.full((16,), ps, jnp.int32))
                for j in range(k):
                    pltpu.sync_copy(v_v.at[pl.ds(j * 128, 128)],
                                    acc_sh.at[ai_refs[j]], add=True)
                return carry

            lax.fori_loop(0, n_ch, body, 0)
            plsc.subcore_barrier()

            off = 0
            while off < orows:
                sz = min(ch, orows - off)
                pltpu.sync_copy(acc_sh.at[pl.ds(sid * orows + off, sz)],
                                out_hbm.at[pl.ds(base + sid * orows + off, sz)])
                off += sz
            if p + 1 < pc:
                plsc.subcore_barrier()

    return kern(m_p, dst2d)


def _prep_edges(ei, ea):
    e = ea.shape[0]
    ch, epad = _pick_ch_epad(e)
    src = ei[0].astype(jnp.int32)
    dst = ei[1].astype(jnp.int32)
    src2d = jnp.pad(src, (0, epad - e)).reshape(epad // 128, 128)
    dst2d = jnp.pad(dst, (0, epad - e)).reshape(epad // 128, 128)
    dst2d_s = jnp.pad(dst, (0, epad - e),
                      constant_values=1 << 29).reshape(epad // 128, 128)
    ea_p = _pad_rows(ea, epad)
    return dict(e=e, ch=ch, epad=epad, src2d=src2d, dst2d=dst2d,
                dst2d_s=dst2d_s, ea_p=ea_p, dst=dst)


def _inet(p, send_p, rec_p, ep, alpha, resid_p=None, norm=None):
    w1, b1 = p['edge_mlp']['linears'][0]
    w2, b2 = p['edge_mlp']['linears'][1]
    w3, b3 = p['edge_mlp']['linears'][2]
    gam, bet = p['edge_mlp']['ln']
    if send_p is rec_p:
        s_tab = r_tab = _proj_pair(send_p, w1[H:2 * H], w1[2 * H:])
    else:
        s_tab = _proj_pair(send_p, w1[H:2 * H], w1[H:2 * H])
        r_tab = _proj_pair(rec_p, w1[2 * H:], w1[2 * H:])
    g = _sc_gather_sum(s_tab, r_tab, ep['src2d'], ep['dst2d'], ep['ch'])
    m_edges = _edge_mlp(ep['ea_p'], g, w1[:H], b1, w2, b2, w3, b3,
                        gam, bet)
    aggr_p = jax.ops.segment_sum(m_edges[:ep['e']], ep['dst'],
                                 num_segments=rec_p.shape[0])
    wn1, bn1 = p['node_mlp']['linears'][0]
    wn2, bn2 = p['node_mlp']['linears'][1]
    wn3, bn3 = p['node_mlp']['linears'][2]
    ngam, nbet = p['node_mlp']['ln']
    return _node_mlp(rec_p, aggr_p, wn1[:H], wn1[H:], bn1, wn2, bn2,
                     wn3, bn3, ngam, nbet, alpha, resid_p, norm)


NUM_LEVELS_K = 3
NUM_STEPS_K = 2


def kernel(mesh_features_list, mesh_edge_index_list, mesh_edge_attr_list,
           up_edge_index_list, up_edge_attr_list, down_edge_index_list,
           down_edge_attr_list, params):
    n_sizes = [f.shape[0] for f in mesh_features_list]
    feats = [_pad_rows(f, BM) for f in mesh_features_list]
    for level in range(NUM_LEVELS_K):
        residual = feats[level]
        ep = _prep_edges(mesh_edge_index_list[level],
                         mesh_edge_attr_list[level])
        for step in range(NUM_STEPS_K):
            last = step == NUM_STEPS_K - 1
            feats[level] = _inet(
                params['intra'][level][step], feats[level], feats[level],
                ep, alpha=1.0,
                resid_p=residual if last else None,
                norm=params['norms'][level] if last else None)
    for level in range(NUM_LEVELS_K - 1):
        ep = _prep_edges(up_edge_index_list[level], up_edge_attr_list[level])
        feats[level + 1] = _inet(
            params['up'][level], feats[level], feats[level + 1], ep,
            alpha=2.0)
    for level in reversed(range(NUM_LEVELS_K - 1)):
        ep = _prep_edges(down_edge_index_list[level],
                         down_edge_attr_list[level])
        feats[level] = _inet(
            params['down'][level], feats[level + 1], feats[level], ep,
            alpha=2.0)
    return tuple(feats[l][:n_sizes[l]] for l in range(NUM_LEVELS_K))
